# baseline (speedup 1.0000x reference)
import jax
import jax.numpy as jnp
from jax.experimental import pallas as pl
from jax.experimental.pallas import tpu as pltpu

HIDDEN = 256
EPS = 1e-6
KCHUNK = 4


def _block_kernel(x_ref, p_ref, w_ref, b_ref, g1_ref, b1_ref, g2_ref, b2_ref,
                  o_ref, acc_ref, xbuf_ref):
    k = pl.program_id(1)
    nk = pl.num_programs(1)
    xk = x_ref[0]
    tk = xk.shape[0]

    mu = jnp.mean(xk, axis=-1, keepdims=True)
    xc = xk - mu
    var = jnp.mean(xc * xc, axis=-1, keepdims=True)
    h = g1_ref[0] * xc / jnp.sqrt(var + EPS) + b1_ref[0]

    partial = jnp.dot(p_ref[0], h, preferred_element_type=jnp.float32)

    @pl.when(k == 0)
    def _init():
        acc_ref[...] = partial

    @pl.when(k > 0)
    def _acc():
        acc_ref[...] += partial

    xbuf_ref[pl.ds(k * tk, tk), :] = xk

    @pl.when(k == nk - 1)
    def _finish():
        conv = jnp.maximum(
            jnp.dot(acc_ref[...], w_ref[...],
                    preferred_element_type=jnp.float32) + b_ref[0], 0.0)
        y = xbuf_ref[...] + conv
        mu2 = jnp.mean(y, axis=-1, keepdims=True)
        yc = y - mu2
        var2 = jnp.mean(yc * yc, axis=-1, keepdims=True)
        o_ref[0] = g2_ref[0] * yc / jnp.sqrt(var2 + EPS) + b2_ref[0]


def kernel(x, mask, inputP, W, b, ln1_g, ln1_b, ln2_g, ln2_b):
    del mask
    B, N, H = x.shape
    TK = N // KCHUNK

    vec = lambda v: v.reshape(1, H)
    return pl.pallas_call(
        _block_kernel,
        grid=(B, KCHUNK),
        in_specs=[
            pl.BlockSpec((1, TK, H), lambda i, k: (i, k, 0)),
            pl.BlockSpec((1, N, TK), lambda i, k: (i, 0, k)),
            pl.BlockSpec((H, H), lambda i, k: (0, 0)),
            pl.BlockSpec((1, H), lambda i, k: (0, 0)),
            pl.BlockSpec((1, H), lambda i, k: (0, 0)),
            pl.BlockSpec((1, H), lambda i, k: (0, 0)),
            pl.BlockSpec((1, H), lambda i, k: (0, 0)),
            pl.BlockSpec((1, H), lambda i, k: (0, 0)),
        ],
        out_specs=pl.BlockSpec((1, N, H), lambda i, k: (i, 0, 0)),
        out_shape=jax.ShapeDtypeStruct((B, N, H), x.dtype),
        scratch_shapes=[pltpu.VMEM((N, H), jnp.float32),
                        pltpu.VMEM((N, H), jnp.float32)],
        compiler_params=pltpu.CompilerParams(
            dimension_semantics=("arbitrary", "arbitrary")),
    )(x, inputP, W, vec(b), vec(ln1_g), vec(ln1_b), vec(ln2_g), vec(ln2_b))


# speedup vs baseline: 2.0982x; 2.0982x over previous
import jax
import jax.numpy as jnp
from jax.experimental import pallas as pl
from jax.experimental.pallas import tpu as pltpu

HIDDEN = 256
EPS = 1e-6


def _block_kernel(x_ref, p_ref, w_ref, b_ref, g1_ref, b1_ref, g2_ref, b2_ref,
                  o_ref):
    x = x_ref[0]
    p = p_ref[0]

    mu = jnp.mean(x, axis=-1, keepdims=True)
    xc = x - mu
    var = jnp.mean(xc * xc, axis=-1, keepdims=True)
    h = g1_ref[0] * xc / jnp.sqrt(var + EPS) + b1_ref[0]

    agg = jnp.dot(p, h, preferred_element_type=jnp.float32)
    conv = jnp.maximum(
        jnp.dot(agg, w_ref[...], preferred_element_type=jnp.float32)
        + b_ref[0], 0.0)

    y = x + conv
    mu2 = jnp.mean(y, axis=-1, keepdims=True)
    yc = y - mu2
    var2 = jnp.mean(yc * yc, axis=-1, keepdims=True)
    o_ref[0] = g2_ref[0] * yc / jnp.sqrt(var2 + EPS) + b2_ref[0]


def kernel(x, mask, inputP, W, b, ln1_g, ln1_b, ln2_g, ln2_b):
    del mask
    B, N, H = x.shape

    vec = lambda v: v.reshape(1, H)
    return pl.pallas_call(
        _block_kernel,
        grid=(B,),
        in_specs=[
            pl.BlockSpec((1, N, H), lambda i: (i, 0, 0)),
            pl.BlockSpec((1, N, N), lambda i: (i, 0, 0)),
            pl.BlockSpec((H, H), lambda i: (0, 0)),
            pl.BlockSpec((1, H), lambda i: (0, 0)),
            pl.BlockSpec((1, H), lambda i: (0, 0)),
            pl.BlockSpec((1, H), lambda i: (0, 0)),
            pl.BlockSpec((1, H), lambda i: (0, 0)),
            pl.BlockSpec((1, H), lambda i: (0, 0)),
        ],
        out_specs=pl.BlockSpec((1, N, H), lambda i: (i, 0, 0)),
        out_shape=jax.ShapeDtypeStruct((B, N, H), x.dtype),
        compiler_params=pltpu.CompilerParams(
            dimension_semantics=("parallel",)),
    )(x, inputP, W, vec(b), vec(ln1_g), vec(ln1_b), vec(ln2_g), vec(ln2_b))


# speedup vs baseline: 2.2078x; 1.0522x over previous
import jax
import jax.numpy as jnp
from jax.experimental import pallas as pl
from jax.experimental.pallas import tpu as pltpu

HIDDEN = 256
EPS = 1e-6


def _block_kernel(x_ref, p_ref, w_ref, b_ref, g1_ref, b1_ref, g2_ref, b2_ref,
                  o_ref):
    x = x_ref[0]
    p = p_ref[0]
    inv_h = 1.0 / x.shape[-1]

    mu = jnp.sum(x, axis=-1, keepdims=True) * inv_h
    m2 = jnp.sum(x * x, axis=-1, keepdims=True) * inv_h
    r = jax.lax.rsqrt(m2 - mu * mu + EPS)
    h = ((x - mu) * r) * g1_ref[0] + b1_ref[0]

    agg = jnp.dot(p, h, preferred_element_type=jnp.float32)
    conv = jnp.maximum(
        jnp.dot(agg, w_ref[...], preferred_element_type=jnp.float32)
        + b_ref[0], 0.0)

    y = x + conv
    mu2 = jnp.sum(y, axis=-1, keepdims=True) * inv_h
    n2 = jnp.sum(y * y, axis=-1, keepdims=True) * inv_h
    r2 = jax.lax.rsqrt(n2 - mu2 * mu2 + EPS)
    o_ref[0] = ((y - mu2) * r2) * g2_ref[0] + b2_ref[0]


def kernel(x, mask, inputP, W, b, ln1_g, ln1_b, ln2_g, ln2_b):
    del mask
    B, N, H = x.shape

    vec = lambda v: v.reshape(1, H)
    return pl.pallas_call(
        _block_kernel,
        grid=(B,),
        in_specs=[
            pl.BlockSpec((1, N, H), lambda i: (i, 0, 0)),
            pl.BlockSpec((1, N, N), lambda i: (i, 0, 0)),
            pl.BlockSpec((H, H), lambda i: (0, 0)),
            pl.BlockSpec((1, H), lambda i: (0, 0)),
            pl.BlockSpec((1, H), lambda i: (0, 0)),
            pl.BlockSpec((1, H), lambda i: (0, 0)),
            pl.BlockSpec((1, H), lambda i: (0, 0)),
            pl.BlockSpec((1, H), lambda i: (0, 0)),
        ],
        out_specs=pl.BlockSpec((1, N, H), lambda i: (i, 0, 0)),
        out_shape=jax.ShapeDtypeStruct((B, N, H), x.dtype),
        compiler_params=pltpu.CompilerParams(
            dimension_semantics=("parallel",)),
    )(x, inputP, W, vec(b), vec(ln1_g), vec(ln1_b), vec(ln2_g), vec(ln2_b))
